# baseline (speedup 1.0000x reference)
import functools

import jax
import jax.numpy as jnp
from jax import lax
from jax.experimental import pallas as pl
from jax.experimental.pallas import tpu as pltpu

_LN_EPS = 1e-5
_LN_TILE = 1024
_Q_TILE = 256
_COL_CHUNK = 256


def _layernorm_kernel(embed_ref, gamma_ref, beta_ref, nodevec_ref):
    x = embed_ref[...]
    mean = jnp.mean(x, axis=-1, keepdims=True)
    centered = x - mean
    var = jnp.mean(centered * centered, axis=-1, keepdims=True)
    nv = centered * lax.rsqrt(var + _LN_EPS)
    nv = nv * gamma_ref[...] + beta_ref[...]
    nodevec_ref[...] = nv.astype(nodevec_ref.dtype)


def _layernorm(embed, gamma, beta):
    n, e = embed.shape
    ln_tile = min(_LN_TILE, n)
    return pl.pallas_call(
        _layernorm_kernel,
        out_shape=jax.ShapeDtypeStruct((n, e), jnp.bfloat16),
        grid=(pl.cdiv(n, ln_tile),),
        in_specs=[
            pl.BlockSpec((ln_tile, e), lambda i: (i, 0)),
            pl.BlockSpec((1, e), lambda i: (0, 0)),
            pl.BlockSpec((1, e), lambda i: (0, 0)),
        ],
        out_specs=pl.BlockSpec((ln_tile, e), lambda i: (i, 0)),
        compiler_params=pltpu.CompilerParams(
            dimension_semantics=("parallel",),
        ),
    )(embed, gamma, beta)


def _chunk_scores(q, k_ref, c, cw):
    kc = k_ref[pl.ds(c * cw, cw), :]
    sc = lax.dot_general(
        q, kc,
        dimension_numbers=(((1,), (1,)), ((), ())),
        preferred_element_type=jnp.float32,
    )
    return jnp.maximum(sc, 0.0)


def _adjacency_flash_kernel(cw, q_ref, k_ref, adj_ref):
    q = q_ref[...]
    nk = k_ref.shape[0]
    nchunks = nk // cw

    m = None
    l = None
    chunk_maxes = []
    for c in range(nchunks):
        sc = _chunk_scores(q, k_ref, c, cw)
        mc = jnp.max(sc, axis=-1, keepdims=True)
        if m is None:
            m = mc
            pc = jnp.exp(sc - m)
            l = jnp.sum(pc, axis=-1, keepdims=True)
        else:
            m_new = jnp.maximum(m, mc)
            pc = jnp.exp(sc - m_new)
            l = l * jnp.exp(m - m_new) + jnp.sum(pc, axis=-1, keepdims=True)
            m = m_new
        chunk_maxes.append(m)
        adj_ref[:, pl.ds(c * cw, cw)] = pc

    r = pl.reciprocal(l, approx=True)
    for c in range(nchunks):
        factor = jnp.exp(chunk_maxes[c] - m) * r
        sl = pl.ds(c * cw, cw)
        adj_ref[:, sl] = adj_ref[:, sl] * factor


def _adjacency_simple_kernel(q_ref, k_ref, adj_ref):
    scores = lax.dot_general(
        q_ref[...], k_ref[...],
        dimension_numbers=(((1,), (1,)), ((), ())),
        preferred_element_type=jnp.float32,
    )
    s = jnp.maximum(scores, 0.0)
    m = jnp.max(s, axis=-1, keepdims=True)
    p = jnp.exp(s - m)
    denom = jnp.sum(p, axis=-1, keepdims=True)
    adj_ref[...] = p * pl.reciprocal(denom, approx=True)


def _adjacency(nodevec):
    n, e = nodevec.shape
    tq = min(_Q_TILE, n)
    if n % _COL_CHUNK == 0 and n // _COL_CHUNK >= 2:
        body = functools.partial(_adjacency_flash_kernel, _COL_CHUNK)
    else:
        body = _adjacency_simple_kernel
    return pl.pallas_call(
        body,
        out_shape=jax.ShapeDtypeStruct((n, n), jnp.float32),
        grid=(pl.cdiv(n, tq),),
        in_specs=[
            pl.BlockSpec((tq, e), lambda i: (i, 0)),
            pl.BlockSpec((n, e), lambda i: (0, 0)),
        ],
        out_specs=pl.BlockSpec((tq, n), lambda i: (i, 0)),
        compiler_params=pltpu.CompilerParams(
            dimension_semantics=("parallel",),
        ),
    )(nodevec, nodevec)


def kernel(embed, ln_weight, ln_bias):
    num_nodes, embed_dim = embed.shape
    gamma = ln_weight.reshape(1, embed_dim).astype(jnp.float32)
    beta = ln_bias.reshape(1, embed_dim).astype(jnp.float32)
    nodevec = _layernorm(embed, gamma, beta)
    return _adjacency(nodevec)


# speedup vs baseline: 1.2162x; 1.2162x over previous
import functools

import jax
import jax.numpy as jnp
from jax import lax
from jax.experimental import pallas as pl
from jax.experimental.pallas import tpu as pltpu

_LN_EPS = 1e-5
_LN_TILE = 1024
_Q_TILE = 256
_COL_CHUNK = 1024


def _layernorm_kernel(embed_ref, gamma_ref, beta_ref, nodevec_ref):
    x = embed_ref[...]
    mean = jnp.mean(x, axis=-1, keepdims=True)
    centered = x - mean
    var = jnp.mean(centered * centered, axis=-1, keepdims=True)
    nv = centered * lax.rsqrt(var + _LN_EPS)
    nv = nv * gamma_ref[...] + beta_ref[...]
    nodevec_ref[...] = nv.astype(nodevec_ref.dtype)


def _layernorm(embed, gamma, beta):
    n, e = embed.shape
    ln_tile = min(_LN_TILE, n)
    return pl.pallas_call(
        _layernorm_kernel,
        out_shape=jax.ShapeDtypeStruct((n, e), jnp.bfloat16),
        grid=(pl.cdiv(n, ln_tile),),
        in_specs=[
            pl.BlockSpec((ln_tile, e), lambda i: (i, 0)),
            pl.BlockSpec((1, e), lambda i: (0, 0)),
            pl.BlockSpec((1, e), lambda i: (0, 0)),
        ],
        out_specs=pl.BlockSpec((ln_tile, e), lambda i: (i, 0)),
        compiler_params=pltpu.CompilerParams(
            dimension_semantics=("parallel",),
        ),
    )(embed, gamma, beta)


def _chunk_scores(q, k_ref, c, cw):
    kc = k_ref[pl.ds(c * cw, cw), :]
    sc = lax.dot_general(
        q, kc,
        dimension_numbers=(((1,), (1,)), ((), ())),
        preferred_element_type=jnp.float32,
    )
    return jnp.maximum(sc, 0.0)


def _adjacency_flash_kernel(cw, q_ref, k_ref, adj_ref):
    q = q_ref[...]
    nk = k_ref.shape[0]
    nchunks = nk // cw

    m = None
    l = None
    chunk_maxes = []
    for c in range(nchunks):
        sc = _chunk_scores(q, k_ref, c, cw)
        mc = jnp.max(sc, axis=-1, keepdims=True)
        if m is None:
            m = mc
            pc = jnp.exp(sc - m)
            l = jnp.sum(pc, axis=-1, keepdims=True)
        else:
            m_new = jnp.maximum(m, mc)
            pc = jnp.exp(sc - m_new)
            l = l * jnp.exp(m - m_new) + jnp.sum(pc, axis=-1, keepdims=True)
            m = m_new
        chunk_maxes.append(m)
        adj_ref[:, pl.ds(c * cw, cw)] = pc

    r = pl.reciprocal(l, approx=True)
    for c in range(nchunks):
        factor = jnp.exp(chunk_maxes[c] - m) * r
        sl = pl.ds(c * cw, cw)
        adj_ref[:, sl] = adj_ref[:, sl] * factor


def _adjacency_simple_kernel(q_ref, k_ref, adj_ref):
    scores = lax.dot_general(
        q_ref[...], k_ref[...],
        dimension_numbers=(((1,), (1,)), ((), ())),
        preferred_element_type=jnp.float32,
    )
    s = jnp.maximum(scores, 0.0)
    m = jnp.max(s, axis=-1, keepdims=True)
    p = jnp.exp(s - m)
    denom = jnp.sum(p, axis=-1, keepdims=True)
    adj_ref[...] = p * pl.reciprocal(denom, approx=True)


def _adjacency(nodevec):
    n, e = nodevec.shape
    tq = min(_Q_TILE, n)
    if n % _COL_CHUNK == 0 and n // _COL_CHUNK >= 2:
        body = functools.partial(_adjacency_flash_kernel, _COL_CHUNK)
    else:
        body = _adjacency_simple_kernel
    return pl.pallas_call(
        body,
        out_shape=jax.ShapeDtypeStruct((n, n), jnp.float32),
        grid=(pl.cdiv(n, tq),),
        in_specs=[
            pl.BlockSpec((tq, e), lambda i: (i, 0)),
            pl.BlockSpec((n, e), lambda i: (0, 0)),
        ],
        out_specs=pl.BlockSpec((tq, n), lambda i: (i, 0)),
        compiler_params=pltpu.CompilerParams(
            dimension_semantics=("parallel",),
        ),
    )(nodevec, nodevec)


def kernel(embed, ln_weight, ln_bias):
    num_nodes, embed_dim = embed.shape
    gamma = ln_weight.reshape(1, embed_dim).astype(jnp.float32)
    beta = ln_bias.reshape(1, embed_dim).astype(jnp.float32)
    nodevec = _layernorm(embed, gamma, beta)
    return _adjacency(nodevec)
